# baseline (speedup 1.0000x reference)
import functools

import jax
import jax.numpy as jnp
from jax import lax
from jax.experimental import pallas as pl
from jax.experimental.pallas import tpu as pltpu
from jax.experimental.pallas import tpu_sc as plsc

VOCAB = 100000
EMB = 64
CTX = 200
LANES = 16

CHUNK = 400
N_GROUPS = CHUNK // LANES
N_CHUNKS = VOCAB // CHUNK
N_WORKERS = 32


def _body(idx_hbm, wproj_hbm, wpred_hbm, out_hbm,
          idx_v, rows_v, wbuf_v, bcast_v, outbuf_v, sem):
    nc = lax.axis_size("c")
    wid = lax.axis_index("s") * nc + lax.axis_index("c")

    pltpu.sync_copy(idx_hbm, idx_v)
    pltpu.async_copy(wproj_hbm.at[idx_v.at[0]], rows_v.at[pl.ds(0, 100)], sem).wait()
    pltpu.async_copy(wproj_hbm.at[idx_v.at[1]], rows_v.at[pl.ds(100, 100)], sem).wait()

    zero = jnp.zeros((LANES,), jnp.float32)

    def row_sum(r, accs):
        return tuple(accs[g] + rows_v[r, pl.ds(g * LANES, LANES)]
                     for g in range(EMB // LANES))

    embs = lax.fori_loop(0, CTX, row_sum, (zero,) * (EMB // LANES))

    for g in range(EMB // LANES):
        for l in range(LANES):
            bcast_v[g * LANES + l, :] = jnp.full((LANES,), embs[g][l],
                                                 dtype=jnp.float32)

    def do_chunk(j, carry):
        c = wid + N_WORKERS * j

        @pl.when(c < N_CHUNKS)
        def _():
            start = pl.multiple_of(c * CHUNK, CHUNK)
            pltpu.sync_copy(wpred_hbm.at[:, pl.ds(start, CHUNK)], wbuf_v)

            def k_step(k, accs):
                bc = bcast_v[k, :]
                return tuple(accs[g] + bc * wbuf_v[k, pl.ds(g * LANES, LANES)]
                             for g in range(N_GROUPS))

            accs = lax.fori_loop(0, EMB, k_step, (zero,) * N_GROUPS)
            for g in range(N_GROUPS):
                outbuf_v[pl.ds(g * LANES, LANES)] = accs[g]
            pltpu.sync_copy(outbuf_v, out_hbm.at[0, pl.ds(start, CHUNK)])

        return carry

    lax.fori_loop(0, (N_CHUNKS + N_WORKERS - 1) // N_WORKERS, do_chunk, 0)


def kernel(input, W_proj, W_pred):
    idx = input.astype(jnp.int32).reshape(2, 100)
    mesh = plsc.VectorSubcoreMesh(core_axis_name="c", subcore_axis_name="s")
    run = functools.partial(
        pl.kernel,
        out_type=jax.ShapeDtypeStruct((1, VOCAB), jnp.float32),
        mesh=mesh,
        compiler_params=pltpu.CompilerParams(use_tc_tiling_on_sc=False),
        scratch_types=[
            pltpu.VMEM((2, 100), jnp.int32),
            pltpu.VMEM((CTX, EMB), jnp.float32),
            pltpu.VMEM((EMB, CHUNK), jnp.float32),
            pltpu.VMEM((EMB, LANES), jnp.float32),
            pltpu.VMEM((CHUNK,), jnp.float32),
            pltpu.SemaphoreType.DMA,
        ],
    )(_body)
    return run(idx, W_proj, W_pred)


# speedup vs baseline: 1.2572x; 1.2572x over previous
import functools

import jax
import jax.numpy as jnp
from jax import lax
from jax.experimental import pallas as pl
from jax.experimental.pallas import tpu as pltpu
from jax.experimental.pallas import tpu_sc as plsc

VOCAB = 100000
EMB = 64
CTX = 200
LANES = 16

BLK = 4096


def _gather_body(idx_hbm, wproj_hbm, out_hbm, idx_v, rows_v, emb_v, sem):
    nc = lax.axis_size("c")
    wid = lax.axis_index("s") * nc + lax.axis_index("c")

    @pl.when(wid == 0)
    def _():
        pltpu.sync_copy(idx_hbm, idx_v)
        pltpu.async_copy(wproj_hbm.at[idx_v.at[0]],
                         rows_v.at[pl.ds(0, 100)], sem).wait()
        pltpu.async_copy(wproj_hbm.at[idx_v.at[1]],
                         rows_v.at[pl.ds(100, 100)], sem).wait()

        for g in range(EMB // LANES):
            acc = rows_v[0, pl.ds(g * LANES, LANES)]
            for r in range(1, CTX):
                acc = acc + rows_v[r, pl.ds(g * LANES, LANES)]
            emb_v[pl.ds(g * LANES, LANES)] = acc
        pltpu.sync_copy(emb_v, out_hbm)


def _sc_gather_sum(idx, W_proj):
    mesh = plsc.VectorSubcoreMesh(core_axis_name="c", subcore_axis_name="s")
    run = functools.partial(
        pl.kernel,
        out_type=jax.ShapeDtypeStruct((EMB,), jnp.float32),
        mesh=mesh,
        compiler_params=pltpu.CompilerParams(use_tc_tiling_on_sc=False),
        scratch_types=[
            pltpu.VMEM((2, 100), jnp.int32),
            pltpu.VMEM((CTX, EMB), jnp.float32),
            pltpu.VMEM((EMB,), jnp.float32),
            pltpu.SemaphoreType.DMA,
        ],
    )(_gather_body)
    return run(idx, W_proj)


def _matmul_body(emb_ref, w_ref, o_ref):
    o_ref[...] = jnp.dot(emb_ref[...], w_ref[...],
                         preferred_element_type=jnp.float32)


def _tc_project(emb, W_pred):
    grid = (VOCAB + BLK - 1) // BLK
    return pl.pallas_call(
        _matmul_body,
        grid=(grid,),
        in_specs=[
            pl.BlockSpec((1, EMB), lambda i: (0, 0)),
            pl.BlockSpec((EMB, BLK), lambda i: (0, i)),
        ],
        out_specs=pl.BlockSpec((1, BLK), lambda i: (0, i)),
        out_shape=jax.ShapeDtypeStruct((1, VOCAB), jnp.float32),
    )(emb, W_pred)


def kernel(input, W_proj, W_pred):
    idx = input.astype(jnp.int32).reshape(2, 100)
    emb = _sc_gather_sum(idx, W_proj).reshape(1, EMB)
    return _tc_project(emb, W_pred)
